# Initial kernel scaffold; baseline (speedup 1.0000x reference)
import jax
import jax.numpy as jnp
from jax.experimental import pallas as pl


def kernel(PFN_output, pillar_tensor, batch_size):
    raise NotImplementedError("write your pallas kernel here")


# speedup vs baseline: 3.1388x; 3.1388x over previous
import jax
import jax.numpy as jnp
from jax import lax
from jax.experimental import pallas as pl
from jax.experimental.pallas import tpu as pltpu
from jax.experimental.pallas import tpu_sc as plsc

C = 64
H = 496
W = 432
HW = H * W
P = 40000
B = 4
N = B * HW
NC = 2
NS = 16
NW = NC * NS
RANGE = N // NW
KCH = 8000
NKCH = P // KCH
CW = 128
T = 3456
GPB = HW // T
L = 16
DUMP = N

_SC_PARAMS = pltpu.CompilerParams(needs_layout_passes=False)


def _k1_body(key_hbm, ptr_hbm, keys_v, lmap):
  wid = lax.axis_index("s") * NC + lax.axis_index("c")
  kbase = wid * RANGE
  iota = lax.iota(jnp.int32, L)

  def zero_body(i, _):
    lmap[pl.ds(i * L, L)] = jnp.zeros((L,), jnp.int32)
    return 0
  lax.fori_loop(0, RANGE // L, zero_body, 0)

  for ci in range(NKCH):
    pltpu.sync_copy(key_hbm.at[pl.ds(ci * KCH, KCH)], keys_v)

    def scan_body(i, _, ci=ci):
      k16 = keys_v[pl.ds(i * L, L)]
      pv = ci * KCH + i * L + iota + 1
      inr = (k16 >= kbase) & (k16 < kbase + RANGE)
      kk = jnp.where(inr, k16 - kbase, 0)
      plsc.store_scatter(lmap, [kk], pv, mask=inr)
      g = plsc.load_gather(lmap, [kk])
      lost = inr & (g < pv)

      def fix(lost0):
        def cond(lost_c):
          return jnp.any(lost_c)

        def step(lost_c):
          plsc.store_scatter(lmap, [kk], pv, mask=lost_c)
          g2 = plsc.load_gather(lmap, [kk])
          return inr & (g2 < pv)
        lax.while_loop(cond, step, lost0)
        return 0
      lax.cond(jnp.any(lost), fix, lambda _: 0, lost)
      return 0
    lax.fori_loop(0, KCH // L, scan_body, 0)

  pltpu.sync_copy(lmap, ptr_hbm.at[pl.ds(kbase, RANGE)])


def _k2_body(feat_hbm, key_hbm, ptr_hbm, canvas_hbm,
             keys_c, w_c, ridx_c, ridx_t, featbuf, sem_k, sem_f, sem_g, sem_s):
  wid = lax.axis_index("s") * NC + lax.axis_index("c")
  iota = lax.iota(jnp.int32, L)

  def chunk(base, cs, keys_ref, ridx_ref):
    nv = cs // L
    pltpu.async_copy(key_hbm.at[pl.ds(base, cs)], keys_ref, sem_k).wait()
    pltpu.async_copy(feat_hbm.at[pl.ds(base, cs), :],
                     featbuf.at[pl.ds(0, cs), :], sem_f).wait()
    pltpu.async_copy(ptr_hbm.at[keys_ref], w_c.at[pl.ds(0, cs)], sem_g).wait()

    def vbody(v, _):
      k16 = keys_ref[pl.ds(v * L, L)]
      w16 = w_c[pl.ds(v * L, L)]
      p16 = base + v * L + iota + 1
      ridx_ref[pl.ds(v * L, L)] = jnp.where(w16 == p16, k16, DUMP)
      return 0
    lax.fori_loop(0, nv, vbody, 0)
    pltpu.async_copy(featbuf.at[pl.ds(0, cs), :],
                     canvas_hbm.at[ridx_ref], sem_s).wait()

  nfull = jnp.where(wid < NW - 1, 10, 2)

  def cbody(j, _):
    chunk(wid * 1280 + j * 128, 128, keys_c, ridx_c)
    return 0
  lax.fori_loop(0, nfull, cbody, 0)

  @pl.when(wid == NW - 1)
  def _():
    chunk(jnp.int32((NW - 1) * 1280 + 2 * 128), 64, keys_c.at[pl.ds(0, 64)],
          ridx_t)


def _k3_body(canvas_ref, ptr_ref, out_ref):
  cv = canvas_ref[:, :C]
  pt = ptr_ref[0, 0, :]
  out_ref[0] = jnp.where(pt[None, :] > 0, cv.T, jnp.float32(0.0))


@jax.jit
def kernel(PFN_output, pillar_tensor, batch_size):
  del batch_size
  key = (pillar_tensor[:, 0] * HW
         + pillar_tensor[:, 2] * W
         + pillar_tensor[:, 3]).astype(jnp.int32)
  featp = jnp.pad(PFN_output, ((0, 0), (0, CW - C)))

  mesh = plsc.VectorSubcoreMesh(core_axis_name="c", subcore_axis_name="s")
  ptr = pl.kernel(
      _k1_body,
      out_type=jax.ShapeDtypeStruct((N,), jnp.int32),
      mesh=mesh,
      compiler_params=_SC_PARAMS,
      scratch_types=[
          pltpu.VMEM((KCH,), jnp.int32),
          pltpu.VMEM((RANGE,), jnp.int32),
      ],
  )(key)

  canvas = pl.kernel(
      _k2_body,
      out_type=jax.ShapeDtypeStruct((N + 8, CW), jnp.float32),
      mesh=mesh,
      compiler_params=_SC_PARAMS,
      scratch_types=[
          pltpu.VMEM((128,), jnp.int32),
          pltpu.VMEM((128,), jnp.int32),
          pltpu.VMEM((128,), jnp.int32),
          pltpu.VMEM((64,), jnp.int32),
          pltpu.VMEM((128, CW), jnp.float32),
          pltpu.SemaphoreType.DMA,
          pltpu.SemaphoreType.DMA,
          pltpu.SemaphoreType.DMA,
          pltpu.SemaphoreType.DMA,
      ],
  )(featp, key, ptr)

  out = pl.pallas_call(
      _k3_body,
      grid=(B, GPB),
      in_specs=[
          pl.BlockSpec((T, CW), lambda b, t: (b * GPB + t, 0)),
          pl.BlockSpec((1, 1, T), lambda b, t: (b * GPB + t, 0, 0)),
      ],
      out_specs=pl.BlockSpec((1, C, T), lambda b, t: (b, 0, t)),
      out_shape=jax.ShapeDtypeStruct((B, C, HW), jnp.float32),
  )(canvas, ptr.reshape(B * GPB, 1, T))

  return out.reshape(B, C, H, W)


# speedup vs baseline: 6.6826x; 2.1291x over previous
import jax
import jax.numpy as jnp
from jax import lax
from jax.experimental import pallas as pl
from jax.experimental.pallas import tpu as pltpu
from jax.experimental.pallas import tpu_sc as plsc

C = 64
H = 496
W = 432
HW = H * W
P = 40000
B = 4
N = B * HW
NC = 2
NS = 16
NW = NC * NS
RANGE = N // NW
KCH = 8000
NKCH = P // KCH
CW = 128
T = 3456
GPB = HW // T
L = 16
DUMP = N

_SC_PARAMS = pltpu.CompilerParams(needs_layout_passes=False)


def _k1_body(key_hbm, ptr_hbm, keys_v, lmap):
  wid = lax.axis_index("s") * NC + lax.axis_index("c")
  kbase = wid * RANGE
  iota = lax.iota(jnp.int32, L)

  def zero_body(i, _):
    lmap[pl.ds(i * L, L)] = jnp.zeros((L,), jnp.int32)
    return 0
  lax.fori_loop(0, RANGE // L, zero_body, 0)

  for ci in range(NKCH):
    pltpu.sync_copy(key_hbm.at[pl.ds(ci * KCH, KCH)], keys_v)

    def scan_body(i, _, ci=ci):
      k16 = keys_v[pl.ds(i * L, L)]
      pv = ci * KCH + i * L + iota + 1
      inr = (k16 >= kbase) & (k16 < kbase + RANGE)
      kk = jnp.where(inr, k16 - kbase, 0)
      plsc.store_scatter(lmap, [kk], pv, mask=inr)
      g = plsc.load_gather(lmap, [kk])
      lost = inr & (g < pv)

      def fix(lost0):
        def cond(lost_c):
          return jnp.any(lost_c)

        def step(lost_c):
          plsc.store_scatter(lmap, [kk], pv, mask=lost_c)
          g2 = plsc.load_gather(lmap, [kk])
          return inr & (g2 < pv)
        lax.while_loop(cond, step, lost0)
        return 0
      lax.cond(jnp.any(lost), fix, lambda _: 0, lost)
      return 0
    lax.fori_loop(0, KCH // L, scan_body, 0)

  pltpu.sync_copy(lmap, ptr_hbm.at[pl.ds(kbase, RANGE)])


def _k2_body(feat_hbm, key_hbm, ptr_hbm, canvas_hbm,
             keys_c, w_c, ridx_c, ridx_t, featbuf, sem_k, sem_f, sem_g, sem_s):
  wid = lax.axis_index("s") * NC + lax.axis_index("c")
  iota = lax.iota(jnp.int32, L)

  def chunk(base, cs, keys_ref, ridx_ref):
    nv = cs // L
    pltpu.async_copy(key_hbm.at[pl.ds(base, cs)], keys_ref, sem_k).wait()
    pltpu.async_copy(feat_hbm.at[pl.ds(base, cs), :],
                     featbuf.at[pl.ds(0, cs), :], sem_f).wait()
    pltpu.async_copy(ptr_hbm.at[keys_ref], w_c.at[pl.ds(0, cs)], sem_g).wait()

    def vbody(v, _):
      k16 = keys_ref[pl.ds(v * L, L)]
      w16 = w_c[pl.ds(v * L, L)]
      p16 = base + v * L + iota + 1
      ridx_ref[pl.ds(v * L, L)] = jnp.where(w16 == p16, k16, DUMP)
      return 0
    lax.fori_loop(0, nv, vbody, 0)
    pltpu.async_copy(featbuf.at[pl.ds(0, cs), :],
                     canvas_hbm.at[ridx_ref], sem_s).wait()

  nfull = jnp.where(wid < NW - 1, 10, 2)

  def cbody(j, _):
    chunk(wid * 1280 + j * 128, 128, keys_c, ridx_c)
    return 0
  lax.fori_loop(0, nfull, cbody, 0)

  @pl.when(wid == NW - 1)
  def _():
    chunk(jnp.int32((NW - 1) * 1280 + 2 * 128), 64, keys_c.at[pl.ds(0, 64)],
          ridx_t)


def _k3_body(canvas_ref, ptr_ref, out_ref):
  for r in range(T // W):
    cv = canvas_ref[pl.ds(r * W, W), :C]
    pt = ptr_ref[0, 0, pl.ds(r * W, W)]
    out_ref[0, :, r, :] = jnp.where(pt[None, :] > 0, cv.T, jnp.float32(0.0))


@jax.jit
def kernel(PFN_output, pillar_tensor, batch_size):
  del batch_size
  key = (pillar_tensor[:, 0] * HW
         + pillar_tensor[:, 2] * W
         + pillar_tensor[:, 3]).astype(jnp.int32)
  featp = jnp.pad(PFN_output, ((0, 0), (0, CW - C)))

  mesh = plsc.VectorSubcoreMesh(core_axis_name="c", subcore_axis_name="s")
  ptr = pl.kernel(
      _k1_body,
      out_type=jax.ShapeDtypeStruct((N,), jnp.int32),
      mesh=mesh,
      compiler_params=_SC_PARAMS,
      scratch_types=[
          pltpu.VMEM((KCH,), jnp.int32),
          pltpu.VMEM((RANGE,), jnp.int32),
      ],
  )(key)

  canvas = pl.kernel(
      _k2_body,
      out_type=jax.ShapeDtypeStruct((N + 8, CW), jnp.float32),
      mesh=mesh,
      compiler_params=_SC_PARAMS,
      scratch_types=[
          pltpu.VMEM((128,), jnp.int32),
          pltpu.VMEM((128,), jnp.int32),
          pltpu.VMEM((128,), jnp.int32),
          pltpu.VMEM((64,), jnp.int32),
          pltpu.VMEM((128, CW), jnp.float32),
          pltpu.SemaphoreType.DMA,
          pltpu.SemaphoreType.DMA,
          pltpu.SemaphoreType.DMA,
          pltpu.SemaphoreType.DMA,
      ],
  )(featp, key, ptr)

  out = pl.pallas_call(
      _k3_body,
      grid=(B, GPB),
      in_specs=[
          pl.BlockSpec((T, CW), lambda b, t: (b * GPB + t, 0)),
          pl.BlockSpec((1, 1, T), lambda b, t: (b * GPB + t, 0, 0)),
      ],
      out_specs=pl.BlockSpec((1, C, T // W, W), lambda b, t: (b, 0, t, 0)),
      out_shape=jax.ShapeDtypeStruct((B, C, H, W), jnp.float32),
  )(canvas, ptr.reshape(B * GPB, 1, T))

  return out


# speedup vs baseline: 7.5684x; 1.1325x over previous
import jax
import jax.numpy as jnp
from jax import lax
from jax.experimental import pallas as pl
from jax.experimental.pallas import tpu as pltpu
from jax.experimental.pallas import tpu_sc as plsc

C = 64
H = 496
W = 432
WP = 512
HWP = H * WP
P = 40000
B = 4
N = B * HWP
NC = 2
NS = 16
NW = NC * NS
RANGE = N // NW
KCH = 8000
NKCH = P // KCH
CW = 128
RPB = 8
T = RPB * WP
GPB = H // RPB
L = 16
DUMP = N

_SC_PARAMS = pltpu.CompilerParams(needs_layout_passes=False)


def _k1_body(key_hbm, ptr_hbm, keys_v, lmap):
  wid = lax.axis_index("s") * NC + lax.axis_index("c")
  kbase = wid * RANGE
  iota = lax.iota(jnp.int32, L)

  def zero_body(i, _):
    lmap[pl.ds(i * L, L)] = jnp.zeros((L,), jnp.int32)
    return 0
  lax.fori_loop(0, RANGE // L, zero_body, 0)

  for ci in range(NKCH):
    pltpu.sync_copy(key_hbm.at[pl.ds(ci * KCH, KCH)], keys_v)

    def scan_body(i, _, ci=ci):
      k16 = keys_v[pl.ds(i * L, L)]
      pv = ci * KCH + i * L + iota + 1
      inr = (k16 >= kbase) & (k16 < kbase + RANGE)
      kk = jnp.where(inr, k16 - kbase, 0)
      plsc.store_scatter(lmap, [kk], pv, mask=inr)
      g = plsc.load_gather(lmap, [kk])
      lost = inr & (g < pv)

      def fix(lost0):
        def cond(lost_c):
          return jnp.any(lost_c)

        def step(lost_c):
          plsc.store_scatter(lmap, [kk], pv, mask=lost_c)
          g2 = plsc.load_gather(lmap, [kk])
          return inr & (g2 < pv)
        lax.while_loop(cond, step, lost0)
        return 0
      lax.cond(jnp.any(lost), fix, lambda _: 0, lost)
      return 0
    lax.fori_loop(0, KCH // L, scan_body, 0)

  pltpu.sync_copy(lmap, ptr_hbm.at[pl.ds(kbase, RANGE)])


def _k2_body(feat_hbm, key_hbm, ptr_hbm, canvas_hbm,
             keys_c, w_c, ridx_c, ridx_t, featbuf, sem_k, sem_f, sem_g, sem_s):
  wid = lax.axis_index("s") * NC + lax.axis_index("c")
  iota = lax.iota(jnp.int32, L)

  def chunk(base, cs, keys_ref, ridx_ref):
    nv = cs // L
    pltpu.async_copy(key_hbm.at[pl.ds(base, cs)], keys_ref, sem_k).wait()
    pltpu.async_copy(feat_hbm.at[pl.ds(base, cs), :],
                     featbuf.at[pl.ds(0, cs), :], sem_f).wait()
    pltpu.async_copy(ptr_hbm.at[keys_ref], w_c.at[pl.ds(0, cs)], sem_g).wait()

    def vbody(v, _):
      k16 = keys_ref[pl.ds(v * L, L)]
      w16 = w_c[pl.ds(v * L, L)]
      p16 = base + v * L + iota + 1
      ridx_ref[pl.ds(v * L, L)] = jnp.where(w16 == p16, k16, DUMP)
      return 0
    lax.fori_loop(0, nv, vbody, 0)
    pltpu.async_copy(featbuf.at[pl.ds(0, cs), :],
                     canvas_hbm.at[ridx_ref], sem_s).wait()

  nfull = jnp.where(wid < NW - 1, 10, 2)

  def cbody(j, _):
    chunk(wid * 1280 + j * 128, 128, keys_c, ridx_c)
    return 0
  lax.fori_loop(0, nfull, cbody, 0)

  @pl.when(wid == NW - 1)
  def _():
    chunk(jnp.int32((NW - 1) * 1280 + 2 * 128), 64, keys_c.at[pl.ds(0, 64)],
          ridx_t)


def _k3_body(canvas_ref, ptr_ref, out_ref):
  for r in range(RPB):
    cv = canvas_ref[pl.ds(r * WP, WP), :C]
    pt = ptr_ref[0, 0, pl.ds(r * WP, WP)]
    vals = jnp.where(pt[None, :] > 0, cv.T, jnp.float32(0.0))
    out_ref[0, :, r, :] = vals[:, :W]


@jax.jit
def kernel(PFN_output, pillar_tensor, batch_size):
  del batch_size
  key = (pillar_tensor[:, 0] * HWP
         + pillar_tensor[:, 2] * WP
         + pillar_tensor[:, 3]).astype(jnp.int32)
  featp = jnp.pad(PFN_output, ((0, 0), (0, CW - C)))

  mesh = plsc.VectorSubcoreMesh(core_axis_name="c", subcore_axis_name="s")
  ptr = pl.kernel(
      _k1_body,
      out_type=jax.ShapeDtypeStruct((N,), jnp.int32),
      mesh=mesh,
      compiler_params=_SC_PARAMS,
      scratch_types=[
          pltpu.VMEM((KCH,), jnp.int32),
          pltpu.VMEM((RANGE,), jnp.int32),
      ],
  )(key)

  canvas = pl.kernel(
      _k2_body,
      out_type=jax.ShapeDtypeStruct((N + 8, CW), jnp.float32),
      mesh=mesh,
      compiler_params=_SC_PARAMS,
      scratch_types=[
          pltpu.VMEM((128,), jnp.int32),
          pltpu.VMEM((128,), jnp.int32),
          pltpu.VMEM((128,), jnp.int32),
          pltpu.VMEM((64,), jnp.int32),
          pltpu.VMEM((128, CW), jnp.float32),
          pltpu.SemaphoreType.DMA,
          pltpu.SemaphoreType.DMA,
          pltpu.SemaphoreType.DMA,
          pltpu.SemaphoreType.DMA,
      ],
  )(featp, key, ptr)

  out = pl.pallas_call(
      _k3_body,
      grid=(B, GPB),
      in_specs=[
          pl.BlockSpec((T, CW), lambda b, t: (b * GPB + t, 0)),
          pl.BlockSpec((1, 1, T), lambda b, t: (b * GPB + t, 0, 0)),
      ],
      out_specs=pl.BlockSpec((1, C, RPB, W), lambda b, t: (b, 0, t, 0)),
      out_shape=jax.ShapeDtypeStruct((B, C, H, W), jnp.float32),
  )(canvas, ptr.reshape(B * GPB, 1, T))

  return out


# speedup vs baseline: 11.4992x; 1.5194x over previous
import jax
import jax.numpy as jnp
from jax import lax
from jax.experimental import pallas as pl
from jax.experimental.pallas import tpu as pltpu
from jax.experimental.pallas import tpu_sc as plsc

C = 64
H = 496
W = 432
HP = 512
P = 40000
B = 4
N = B * W * HP
NC = 2
NS = 16
NW = NC * NS
RANGE = N // NW
KCH = 8000
NKCH = P // KCH
CW = 128
XPB = 8
T = XPB * HP
GPB = W // XPB
L = 16
DUMP = N

_SC_PARAMS = pltpu.CompilerParams(needs_layout_passes=False)


def _k1_body(key_hbm, ptr_hbm, keys_v, lmap):
  wid = lax.axis_index("s") * NC + lax.axis_index("c")
  kbase = wid * RANGE
  iota = lax.iota(jnp.int32, L)

  def zero_body(i, _):
    lmap[pl.ds(i * L, L)] = jnp.zeros((L,), jnp.int32)
    return 0
  lax.fori_loop(0, RANGE // L, zero_body, 0)

  for ci in range(NKCH):
    pltpu.sync_copy(key_hbm.at[pl.ds(ci * KCH, KCH)], keys_v)

    def scan_body(i, _, ci=ci):
      k16 = keys_v[pl.ds(i * L, L)]
      pv = ci * KCH + i * L + iota + 1
      inr = (k16 >= kbase) & (k16 < kbase + RANGE)
      kk = jnp.where(inr, k16 - kbase, 0)
      plsc.store_scatter(lmap, [kk], pv, mask=inr)
      g = plsc.load_gather(lmap, [kk])
      lost = inr & (g < pv)

      def fix(lost0):
        def cond(lost_c):
          return jnp.any(lost_c)

        def step(lost_c):
          plsc.store_scatter(lmap, [kk], pv, mask=lost_c)
          g2 = plsc.load_gather(lmap, [kk])
          return inr & (g2 < pv)
        lax.while_loop(cond, step, lost0)
        return 0
      lax.cond(jnp.any(lost), fix, lambda _: 0, lost)
      return 0
    lax.fori_loop(0, KCH // L, scan_body, 0)

  pltpu.sync_copy(lmap, ptr_hbm.at[pl.ds(kbase, RANGE)])


def _k2_body(feat_hbm, key_hbm, ptr_hbm, canvas_hbm,
             keys_c, w_c, ridx_c, ridx_t, featbuf, sem_k, sem_f, sem_g, sem_s):
  wid = lax.axis_index("s") * NC + lax.axis_index("c")
  iota = lax.iota(jnp.int32, L)

  def chunk(base, cs, keys_ref, ridx_ref):
    nv = cs // L
    pltpu.async_copy(key_hbm.at[pl.ds(base, cs)], keys_ref, sem_k).wait()
    pltpu.async_copy(feat_hbm.at[pl.ds(base, cs), :],
                     featbuf.at[pl.ds(0, cs), :], sem_f).wait()
    pltpu.async_copy(ptr_hbm.at[keys_ref], w_c.at[pl.ds(0, cs)], sem_g).wait()

    def vbody(v, _):
      k16 = keys_ref[pl.ds(v * L, L)]
      w16 = w_c[pl.ds(v * L, L)]
      p16 = base + v * L + iota + 1
      ridx_ref[pl.ds(v * L, L)] = jnp.where(w16 == p16, k16, DUMP)
      return 0
    lax.fori_loop(0, nv, vbody, 0)
    pltpu.async_copy(featbuf.at[pl.ds(0, cs), :],
                     canvas_hbm.at[ridx_ref], sem_s).wait()

  nfull = jnp.where(wid < NW - 1, 10, 2)

  def cbody(j, _):
    chunk(wid * 1280 + j * 128, 128, keys_c, ridx_c)
    return 0
  lax.fori_loop(0, nfull, cbody, 0)

  @pl.when(wid == NW - 1)
  def _():
    chunk(jnp.int32((NW - 1) * 1280 + 2 * 128), 64, keys_c.at[pl.ds(0, 64)],
          ridx_t)


def _k3_body(canvas_ref, ptr_ref, out_ref):
  for r in range(XPB):
    cv = canvas_ref[pl.ds(r * HP, HP), :C]
    pt = ptr_ref[0, 0, pl.ds(r * HP, HP)]
    vals = jnp.where(pt[None, :] > 0, cv.T, jnp.float32(0.0))
    out_ref[0, :, r, :] = vals[:, :H]


@jax.jit
def kernel(PFN_output, pillar_tensor, batch_size):
  del batch_size
  key = ((pillar_tensor[:, 0] * W + pillar_tensor[:, 3]) * HP
         + pillar_tensor[:, 2]).astype(jnp.int32)
  featp = jnp.pad(PFN_output, ((0, 0), (0, CW - C)))

  mesh = plsc.VectorSubcoreMesh(core_axis_name="c", subcore_axis_name="s")
  ptr = pl.kernel(
      _k1_body,
      out_type=jax.ShapeDtypeStruct((N,), jnp.int32),
      mesh=mesh,
      compiler_params=_SC_PARAMS,
      scratch_types=[
          pltpu.VMEM((KCH,), jnp.int32),
          pltpu.VMEM((RANGE,), jnp.int32),
      ],
  )(key)

  canvas = pl.kernel(
      _k2_body,
      out_type=jax.ShapeDtypeStruct((N + 8, CW), jnp.float32),
      mesh=mesh,
      compiler_params=_SC_PARAMS,
      scratch_types=[
          pltpu.VMEM((128,), jnp.int32),
          pltpu.VMEM((128,), jnp.int32),
          pltpu.VMEM((128,), jnp.int32),
          pltpu.VMEM((64,), jnp.int32),
          pltpu.VMEM((128, CW), jnp.float32),
          pltpu.SemaphoreType.DMA,
          pltpu.SemaphoreType.DMA,
          pltpu.SemaphoreType.DMA,
          pltpu.SemaphoreType.DMA,
      ],
  )(featp, key, ptr)

  out = pl.pallas_call(
      _k3_body,
      grid=(B, GPB),
      in_specs=[
          pl.BlockSpec((T, CW), lambda b, t: (b * GPB + t, 0)),
          pl.BlockSpec((1, 1, T), lambda b, t: (b * GPB + t, 0, 0)),
      ],
      out_specs=pl.BlockSpec((1, C, XPB, H), lambda b, t: (b, 0, t, 0)),
      out_shape=jax.ShapeDtypeStruct((B, C, W, H), jnp.float32),
  )(canvas, ptr.reshape(B * GPB, 1, T))

  return out.transpose(0, 1, 3, 2)


# speedup vs baseline: 13.0522x; 1.1351x over previous
import jax
import jax.numpy as jnp
from jax import lax
from jax.experimental import pallas as pl
from jax.experimental.pallas import tpu as pltpu
from jax.experimental.pallas import tpu_sc as plsc

C = 64
H = 496
W = 432
HP = 512
P = 40000
B = 4
N = B * W * HP
NC = 2
NS = 16
NW = NC * NS
RANGE = N // NW
KCH = 8000
NKCH = P // KCH
CW = 128
XPB = 8
T = XPB * HP
GPB = W // XPB
L = 16
DUMP = N

_SC_PARAMS = pltpu.CompilerParams(needs_layout_passes=False)


def _k1_body(key_hbm, ptr_hbm, keys_v, lmap):
  wid = lax.axis_index("s") * NC + lax.axis_index("c")
  kbase = wid * RANGE
  iota = lax.iota(jnp.int32, L)

  def zero_body(i, _):
    lmap[pl.ds(i * L, L)] = jnp.zeros((L,), jnp.int32)
    return 0
  lax.fori_loop(0, RANGE // L, zero_body, 0)

  GV = 10
  for ci in range(NKCH):
    pltpu.sync_copy(key_hbm.at[pl.ds(ci * KCH, KCH)], keys_v)

    def group_body(gi, _, ci=ci):
      def vreg(u, gi=gi):
        i = gi * GV + u
        k16 = keys_v[pl.ds(i * L, L)]
        pv = ci * KCH + i * L + iota + 1
        inr = (k16 >= kbase) & (k16 < kbase + RANGE)
        kk = jnp.where(inr, k16 - kbase, 0)
        return k16, pv, inr, kk

      acc = jnp.zeros((L,), jnp.bool_)
      for u in range(GV):
        _, pv, inr, kk = vreg(u)
        plsc.store_scatter(lmap, [kk], pv, mask=inr)
        g = plsc.load_gather(lmap, [kk])
        acc = acc | (inr & (g < pv))

      def fix(_):
        for u in range(GV):
          _, pv, inr, kk = vreg(u)

          def cond(lost_c):
            return jnp.any(lost_c)

          def step(lost_c, pv=pv, inr=inr, kk=kk):
            plsc.store_scatter(lmap, [kk], pv, mask=lost_c)
            g2 = plsc.load_gather(lmap, [kk])
            return inr & (g2 < pv)
          g = plsc.load_gather(lmap, [kk])
          lax.while_loop(cond, step, inr & (g < pv))
        return 0
      lax.cond(jnp.any(acc), fix, lambda _: 0, 0)
      return 0
    lax.fori_loop(0, KCH // L // GV, group_body, 0)

  pltpu.sync_copy(lmap, ptr_hbm.at[pl.ds(kbase, RANGE)])


def _k2_body(feat_hbm, key_hbm, ptr_hbm, canvas_hbm,
             keys2, w2, ridx2, featbuf, sem_k, sem_f, sem_g, sem_s):
  wid = lax.axis_index("s") * NC + lax.axis_index("c")
  iota = lax.iota(jnp.int32, L)

  def superchunk(base, rows, tail64):
    kc = []
    for j in range(rows):
      cs = 64 if (tail64 and j == rows - 1) else 128
      kc.append(pltpu.async_copy(key_hbm.at[pl.ds(base + j * 128, cs)],
                                 keys2.at[j, pl.ds(0, cs)], sem_k))
    if tail64:
      for v in range(4):
        keys2[rows - 1, pl.ds(64 + v * L, L)] = jnp.zeros((L,), jnp.int32)
    npil = rows * 128 - (64 if tail64 else 0)
    cf = pltpu.async_copy(feat_hbm.at[pl.ds(base, npil), :],
                          featbuf.at[pl.ds(0, npil), :], sem_f)
    for c in kc:
      c.wait()
    cg = [pltpu.async_copy(ptr_hbm.at[keys2.at[j]], w2.at[j], sem_g)
          for j in range(rows)]
    for c in cg:
      c.wait()
    for j in range(rows):
      for v in range(128 // L):
        k16 = keys2[j, pl.ds(v * L, L)]
        w16 = w2[j, pl.ds(v * L, L)]
        p16 = base + j * 128 + v * L + iota + 1
        ridx2[j, pl.ds(v * L, L)] = jnp.where(w16 == p16, k16, DUMP)
    cf.wait()
    cs = [pltpu.async_copy(featbuf.at[pl.ds(j * 128, 128), :],
                           canvas_hbm.at[ridx2.at[j]], sem_s)
          for j in range(rows)]
    for c in cs:
      c.wait()

  @pl.when(wid < NW - 1)
  def _():
    superchunk(wid * 1280, 5, False)
    superchunk(wid * 1280 + 640, 5, False)

  @pl.when(wid == NW - 1)
  def _():
    superchunk(jnp.int32((NW - 1) * 1280), 3, True)


def _k3_body(canvas_ref, ptr_ref, out_ref):
  for r in range(XPB):
    cv = canvas_ref[pl.ds(r * HP, HP), :C]
    pt = ptr_ref[0, 0, pl.ds(r * HP, HP)]
    vals = jnp.where(pt[None, :] > 0, cv.T, jnp.float32(0.0))
    out_ref[0, :, r, :] = vals[:, :H]


@jax.jit
def kernel(PFN_output, pillar_tensor, batch_size):
  del batch_size
  key = ((pillar_tensor[:, 0] * W + pillar_tensor[:, 3]) * HP
         + pillar_tensor[:, 2]).astype(jnp.int32)
  featp = jnp.pad(PFN_output, ((0, 0), (0, CW - C)))

  mesh = plsc.VectorSubcoreMesh(core_axis_name="c", subcore_axis_name="s")
  ptr = pl.kernel(
      _k1_body,
      out_type=jax.ShapeDtypeStruct((N,), jnp.int32),
      mesh=mesh,
      compiler_params=_SC_PARAMS,
      scratch_types=[
          pltpu.VMEM((KCH,), jnp.int32),
          pltpu.VMEM((RANGE,), jnp.int32),
      ],
  )(key)

  canvas = pl.kernel(
      _k2_body,
      out_type=jax.ShapeDtypeStruct((N + 8, CW), jnp.float32),
      mesh=mesh,
      compiler_params=_SC_PARAMS,
      scratch_types=[
          pltpu.VMEM((5, 128), jnp.int32),
          pltpu.VMEM((5, 128), jnp.int32),
          pltpu.VMEM((5, 128), jnp.int32),
          pltpu.VMEM((640, CW), jnp.float32),
          pltpu.SemaphoreType.DMA,
          pltpu.SemaphoreType.DMA,
          pltpu.SemaphoreType.DMA,
          pltpu.SemaphoreType.DMA,
      ],
  )(featp, key, ptr)

  out = pl.pallas_call(
      _k3_body,
      grid=(B, GPB),
      in_specs=[
          pl.BlockSpec((T, CW), lambda b, t: (b * GPB + t, 0)),
          pl.BlockSpec((1, 1, T), lambda b, t: (b * GPB + t, 0, 0)),
      ],
      out_specs=pl.BlockSpec((1, C, XPB, H), lambda b, t: (b, 0, t, 0)),
      out_shape=jax.ShapeDtypeStruct((B, C, W, H), jnp.float32),
  )(canvas, ptr.reshape(B * GPB, 1, T))

  return out.transpose(0, 1, 3, 2)


# speedup vs baseline: 15.2220x; 1.1662x over previous
import jax
import jax.numpy as jnp
from jax import lax
from jax.experimental import pallas as pl
from jax.experimental.pallas import tpu as pltpu
from jax.experimental.pallas import tpu_sc as plsc

C = 64
H = 496
W = 432
HP = 512
P = 40000
B = 4
N = B * W * HP
NC = 2
NS = 16
NW = NC * NS
RANGE = N // NW
KCH = 8000
NKCH = P // KCH
CW = 128
XPB = 16
T = XPB * HP
GPB = W // XPB
L = 16
DUMP = N

_SC_PARAMS = pltpu.CompilerParams(needs_layout_passes=False)


def _k1_body(key_hbm, ptr_hbm, keys_v, lmap):
  wid = lax.axis_index("s") * NC + lax.axis_index("c")
  kbase = wid * RANGE
  iota = lax.iota(jnp.int32, L)

  def zero_body(i, _):
    lmap[pl.ds(i * L, L)] = jnp.zeros((L,), jnp.int32)
    return 0
  lax.fori_loop(0, RANGE // L, zero_body, 0)

  GV = 10
  for ci in range(NKCH):
    pltpu.sync_copy(key_hbm.at[pl.ds(ci * KCH, KCH)], keys_v)

    def group_body(gi, _, ci=ci):
      def vreg(u, gi=gi):
        i = gi * GV + u
        k16 = keys_v[pl.ds(i * L, L)]
        pv = ci * KCH + i * L + iota + 1
        inr = (k16 >= kbase) & (k16 < kbase + RANGE)
        kk = jnp.where(inr, k16 - kbase, 0)
        return k16, pv, inr, kk

      acc = jnp.zeros((L,), jnp.bool_)
      for u in range(GV):
        _, pv, inr, kk = vreg(u)
        plsc.store_scatter(lmap, [kk], pv, mask=inr)
        g = plsc.load_gather(lmap, [kk])
        acc = acc | (inr & (g < pv))

      def fix(_):
        for u in range(GV):
          _, pv, inr, kk = vreg(u)

          def cond(lost_c):
            return jnp.any(lost_c)

          def step(lost_c, pv=pv, inr=inr, kk=kk):
            plsc.store_scatter(lmap, [kk], pv, mask=lost_c)
            g2 = plsc.load_gather(lmap, [kk])
            return inr & (g2 < pv)
          g = plsc.load_gather(lmap, [kk])
          lax.while_loop(cond, step, inr & (g < pv))
        return 0
      lax.cond(jnp.any(acc), fix, lambda _: 0, 0)
      return 0
    lax.fori_loop(0, KCH // L // GV, group_body, 0)

  pltpu.sync_copy(lmap, ptr_hbm.at[pl.ds(kbase, RANGE)])


def _k2_body(feat_hbm, key_hbm, ptr_hbm, canvas_hbm,
             keys2, w2, ridx2, featbuf, sem_k, sem_f, sem_g, sem_s):
  wid = lax.axis_index("s") * NC + lax.axis_index("c")
  iota = lax.iota(jnp.int32, L)

  def superchunk(base, rows, tail64):
    kc = []
    for j in range(rows):
      cs = 64 if (tail64 and j == rows - 1) else 128
      kc.append(pltpu.async_copy(key_hbm.at[pl.ds(base + j * 128, cs)],
                                 keys2.at[j, pl.ds(0, cs)], sem_k))
    if tail64:
      for v in range(4):
        keys2[rows - 1, pl.ds(64 + v * L, L)] = jnp.zeros((L,), jnp.int32)
    npil = rows * 128 - (64 if tail64 else 0)
    cf = pltpu.async_copy(feat_hbm.at[pl.ds(base, npil), :],
                          featbuf.at[pl.ds(0, npil), :], sem_f)
    for c in kc:
      c.wait()
    cg = [pltpu.async_copy(ptr_hbm.at[keys2.at[j]], w2.at[j], sem_g)
          for j in range(rows)]
    for c in cg:
      c.wait()
    for j in range(rows):
      for v in range(128 // L):
        k16 = keys2[j, pl.ds(v * L, L)]
        w16 = w2[j, pl.ds(v * L, L)]
        p16 = base + j * 128 + v * L + iota + 1
        ridx2[j, pl.ds(v * L, L)] = jnp.where(w16 == p16, k16, DUMP)
    cf.wait()
    cs = [pltpu.async_copy(featbuf.at[pl.ds(j * 128, 128), :],
                           canvas_hbm.at[ridx2.at[j]], sem_s)
          for j in range(rows)]
    for c in cs:
      c.wait()

  @pl.when(wid < NW - 1)
  def _():
    superchunk(wid * 1280, 5, False)
    superchunk(wid * 1280 + 640, 5, False)

  @pl.when(wid == NW - 1)
  def _():
    superchunk(jnp.int32((NW - 1) * 1280), 3, True)


def _k3_body(canvas_ref, ptr_ref, out_ref):
  for r in range(XPB):
    cv = canvas_ref[pl.ds(r * HP, HP), :C]
    pt = ptr_ref[0, 0, pl.ds(r * HP, HP)]
    vals = jnp.where(pt[None, :] > 0, cv.T, jnp.float32(0.0))
    out_ref[0, :, r, :] = vals[:, :H]


@jax.jit
def kernel(PFN_output, pillar_tensor, batch_size):
  del batch_size
  key = ((pillar_tensor[:, 0] * W + pillar_tensor[:, 3]) * HP
         + pillar_tensor[:, 2]).astype(jnp.int32)
  featp = jnp.pad(PFN_output, ((0, 0), (0, CW - C)))

  mesh = plsc.VectorSubcoreMesh(core_axis_name="c", subcore_axis_name="s")
  ptr = pl.kernel(
      _k1_body,
      out_type=jax.ShapeDtypeStruct((N,), jnp.int32),
      mesh=mesh,
      compiler_params=_SC_PARAMS,
      scratch_types=[
          pltpu.VMEM((KCH,), jnp.int32),
          pltpu.VMEM((RANGE,), jnp.int32),
      ],
  )(key)

  canvas = pl.kernel(
      _k2_body,
      out_type=jax.ShapeDtypeStruct((N + 8, CW), jnp.float32),
      mesh=mesh,
      compiler_params=_SC_PARAMS,
      scratch_types=[
          pltpu.VMEM((5, 128), jnp.int32),
          pltpu.VMEM((5, 128), jnp.int32),
          pltpu.VMEM((5, 128), jnp.int32),
          pltpu.VMEM((640, CW), jnp.float32),
          pltpu.SemaphoreType.DMA,
          pltpu.SemaphoreType.DMA,
          pltpu.SemaphoreType.DMA,
          pltpu.SemaphoreType.DMA,
      ],
  )(featp, key, ptr)

  out = pl.pallas_call(
      _k3_body,
      grid=(B, GPB),
      in_specs=[
          pl.BlockSpec((T, CW), lambda b, t: (b * GPB + t, 0)),
          pl.BlockSpec((1, 1, T), lambda b, t: (b * GPB + t, 0, 0)),
      ],
      out_specs=pl.BlockSpec((1, C, XPB, H), lambda b, t: (b, 0, t, 0)),
      out_shape=jax.ShapeDtypeStruct((B, C, W, H), jnp.float32),
  )(canvas, ptr.reshape(B * GPB, 1, T))

  return out.transpose(0, 1, 3, 2)


# speedup vs baseline: 16.0577x; 1.0549x over previous
import jax
import jax.numpy as jnp
from jax import lax
from jax.experimental import pallas as pl
from jax.experimental.pallas import tpu as pltpu
from jax.experimental.pallas import tpu_sc as plsc

C = 64
H = 496
W = 432
HP = 512
P = 40000
B = 4
N = B * W * HP
NC = 2
NS = 16
NW = NC * NS
RANGE = N // NW
KCH = 8000
NKCH = P // KCH
CW = 128
XPB = 24
T = XPB * HP
GPB = W // XPB
L = 16
DUMP = N

_SC_PARAMS = pltpu.CompilerParams(needs_layout_passes=False)


def _k1_body(key_hbm, ptr_hbm, keys_v, lmap):
  wid = lax.axis_index("s") * NC + lax.axis_index("c")
  kbase = wid * RANGE
  iota = lax.iota(jnp.int32, L)

  def zero_body(i, _):
    lmap[pl.ds(i * L, L)] = jnp.zeros((L,), jnp.int32)
    return 0
  lax.fori_loop(0, RANGE // L, zero_body, 0)

  GV = 10
  for ci in range(NKCH):
    pltpu.sync_copy(key_hbm.at[pl.ds(ci * KCH, KCH)], keys_v)

    def group_body(gi, _, ci=ci):
      def vreg(u, gi=gi):
        i = gi * GV + u
        k16 = keys_v[pl.ds(i * L, L)]
        pv = ci * KCH + i * L + iota + 1
        inr = (k16 >= kbase) & (k16 < kbase + RANGE)
        kk = jnp.where(inr, k16 - kbase, 0)
        return k16, pv, inr, kk

      acc = jnp.zeros((L,), jnp.bool_)
      for u in range(GV):
        _, pv, inr, kk = vreg(u)
        plsc.store_scatter(lmap, [kk], pv, mask=inr)
        g = plsc.load_gather(lmap, [kk])
        acc = acc | (inr & (g < pv))

      def fix(_):
        for u in range(GV):
          _, pv, inr, kk = vreg(u)

          def cond(lost_c):
            return jnp.any(lost_c)

          def step(lost_c, pv=pv, inr=inr, kk=kk):
            plsc.store_scatter(lmap, [kk], pv, mask=lost_c)
            g2 = plsc.load_gather(lmap, [kk])
            return inr & (g2 < pv)
          g = plsc.load_gather(lmap, [kk])
          lax.while_loop(cond, step, inr & (g < pv))
        return 0
      lax.cond(jnp.any(acc), fix, lambda _: 0, 0)
      return 0
    lax.fori_loop(0, KCH // L // GV, group_body, 0)

  pltpu.sync_copy(lmap, ptr_hbm.at[pl.ds(kbase, RANGE)])


def _k2_body(feat_hbm, key_hbm, ptr_hbm, canvas_hbm,
             keys2, w2, ridx2, featbuf, sem_k, sem_f, sem_g, sem_s):
  wid = lax.axis_index("s") * NC + lax.axis_index("c")
  iota = lax.iota(jnp.int32, L)

  def superchunk(base, rows, tail64):
    kc = []
    for j in range(rows):
      cs = 64 if (tail64 and j == rows - 1) else 128
      kc.append(pltpu.async_copy(key_hbm.at[pl.ds(base + j * 128, cs)],
                                 keys2.at[j, pl.ds(0, cs)], sem_k))
    if tail64:
      for v in range(4):
        keys2[rows - 1, pl.ds(64 + v * L, L)] = jnp.zeros((L,), jnp.int32)
    npil = rows * 128 - (64 if tail64 else 0)
    cf = pltpu.async_copy(feat_hbm.at[pl.ds(base, npil), :],
                          featbuf.at[pl.ds(0, npil), :], sem_f)
    for c in kc:
      c.wait()
    cg = [pltpu.async_copy(ptr_hbm.at[keys2.at[j]], w2.at[j], sem_g)
          for j in range(rows)]
    for c in cg:
      c.wait()
    for j in range(rows):
      for v in range(128 // L):
        k16 = keys2[j, pl.ds(v * L, L)]
        w16 = w2[j, pl.ds(v * L, L)]
        p16 = base + j * 128 + v * L + iota + 1
        ridx2[j, pl.ds(v * L, L)] = jnp.where(w16 == p16, k16, DUMP)
    cf.wait()
    cs = [pltpu.async_copy(featbuf.at[pl.ds(j * 128, 128), :],
                           canvas_hbm.at[ridx2.at[j]], sem_s)
          for j in range(rows)]
    for c in cs:
      c.wait()

  @pl.when(wid < NW - 1)
  def _():
    superchunk(wid * 1280, 5, False)
    superchunk(wid * 1280 + 640, 5, False)

  @pl.when(wid == NW - 1)
  def _():
    superchunk(jnp.int32((NW - 1) * 1280), 3, True)


def _k3_body(canvas_ref, ptr_ref, out_ref):
  for r in range(XPB):
    cv = canvas_ref[pl.ds(r * HP, HP), :C]
    pt = ptr_ref[0, 0, pl.ds(r * HP, HP)]
    vals = jnp.where(pt[None, :] > 0, cv.T, jnp.float32(0.0))
    out_ref[0, :, r, :] = vals[:, :H]


@jax.jit
def kernel(PFN_output, pillar_tensor, batch_size):
  del batch_size
  key = ((pillar_tensor[:, 0] * W + pillar_tensor[:, 3]) * HP
         + pillar_tensor[:, 2]).astype(jnp.int32)
  featp = jnp.pad(PFN_output, ((0, 0), (0, CW - C)))

  mesh = plsc.VectorSubcoreMesh(core_axis_name="c", subcore_axis_name="s")
  ptr = pl.kernel(
      _k1_body,
      out_type=jax.ShapeDtypeStruct((N,), jnp.int32),
      mesh=mesh,
      compiler_params=_SC_PARAMS,
      scratch_types=[
          pltpu.VMEM((KCH,), jnp.int32),
          pltpu.VMEM((RANGE,), jnp.int32),
      ],
  )(key)

  canvas = pl.kernel(
      _k2_body,
      out_type=jax.ShapeDtypeStruct((N + 8, CW), jnp.float32),
      mesh=mesh,
      compiler_params=_SC_PARAMS,
      scratch_types=[
          pltpu.VMEM((5, 128), jnp.int32),
          pltpu.VMEM((5, 128), jnp.int32),
          pltpu.VMEM((5, 128), jnp.int32),
          pltpu.VMEM((640, CW), jnp.float32),
          pltpu.SemaphoreType.DMA,
          pltpu.SemaphoreType.DMA,
          pltpu.SemaphoreType.DMA,
          pltpu.SemaphoreType.DMA,
      ],
  )(featp, key, ptr)

  out = pl.pallas_call(
      _k3_body,
      grid=(B, GPB),
      in_specs=[
          pl.BlockSpec((T, CW), lambda b, t: (b * GPB + t, 0)),
          pl.BlockSpec((1, 1, T), lambda b, t: (b * GPB + t, 0, 0)),
      ],
      out_specs=pl.BlockSpec((1, C, XPB, H), lambda b, t: (b, 0, t, 0)),
      out_shape=jax.ShapeDtypeStruct((B, C, W, H), jnp.float32),
  )(canvas, ptr.reshape(B * GPB, 1, T))

  return out.transpose(0, 1, 3, 2)
